# baseline (speedup 1.0000x reference)
import functools

import jax
import jax.numpy as jnp
from jax import lax
from jax.experimental import pallas as pl
from jax.experimental.pallas import tpu as pltpu
from jax.experimental.pallas import tpu_sc as plsc

_INFO = plsc.get_sparse_core_info()
NC = _INFO.num_cores
NS = _INFO.num_subcores
NW = NC * NS

CH = 128


@functools.partial(jax.jit, static_argnames=("n_ch", "d"))
def _gather(idx, lut, n_ch, d):
    n = NW * n_ch * CH
    per_w = n_ch * CH
    mesh = plsc.VectorSubcoreMesh(core_axis_name="c", subcore_axis_name="s")

    @functools.partial(
        pl.kernel,
        out_type=jax.ShapeDtypeStruct((n, d), jnp.float32),
        mesh=mesh,
        scratch_types=[
            pltpu.VMEM((n_ch, CH), jnp.int32),
            pltpu.VMEM((CH, d), jnp.float32),
            pltpu.SemaphoreType.DMA,
        ],
        compiler_params=pltpu.CompilerParams(use_tc_tiling_on_sc=False),
    )
    def k(idx_hbm, table_hbm, out_hbm, idx_v, rows_v, sem):
        cid = lax.axis_index("c")
        sid = lax.axis_index("s")
        wid = sid * NC + cid
        pltpu.sync_copy(idx_hbm.at[wid], idx_v)
        base = wid * per_w

        def step(j, carry):
            pltpu.async_copy(table_hbm.at[idx_v.at[j]], rows_v, sem).wait()
            pltpu.sync_copy(rows_v, out_hbm.at[pl.ds(base + j * CH, CH)])
            return carry

        lax.fori_loop(0, n_ch, step, 0)

    return k(idx, lut)


def kernel(x, lut):
    b, h = x.shape
    v, d = lut.shape
    n = b * h
    assert n % (NW * CH) == 0
    n_ch = n // (NW * CH)
    idx = x.reshape(NW, n_ch, CH)
    out = _gather(idx, lut, n_ch, d)
    return out.reshape(b, h, d)


# speedup vs baseline: 1.1148x; 1.1148x over previous
import functools

import jax
import jax.numpy as jnp
from jax import lax
from jax.experimental import pallas as pl
from jax.experimental.pallas import tpu as pltpu
from jax.experimental.pallas import tpu_sc as plsc

_INFO = plsc.get_sparse_core_info()
NC = _INFO.num_cores
NS = _INFO.num_subcores
NW = NC * NS

CH = 128
R = 8
K = 4


@functools.partial(jax.jit, static_argnames=("n_ch", "d"))
def _gather(idx, lut, n_ch, d):
    n = NW * n_ch * CH
    per_w = n_ch * CH
    n_groups = n_ch // R
    assert n_ch % R == 0 and n_groups >= 3
    mesh = plsc.VectorSubcoreMesh(core_axis_name="c", subcore_axis_name="s")

    @functools.partial(
        pl.kernel,
        out_type=jax.ShapeDtypeStruct((n, d), jnp.float32),
        mesh=mesh,
        scratch_types=(
            [pltpu.VMEM((n_ch, CH), jnp.int32), pltpu.VMEM((R, CH, d), jnp.float32)]
            + [pltpu.SemaphoreType.DMA] * (2 * R)
        ),
        compiler_params=pltpu.CompilerParams(use_tc_tiling_on_sc=False),
    )
    def k(idx_hbm, table_hbm, out_hbm, idx_v, rows_v, *sems):
        gsem = sems[:R]
        wsem = sems[R:]
        cid = lax.axis_index("c")
        sid = lax.axis_index("s")
        wid = sid * NC + cid
        pltpu.sync_copy(idx_hbm.at[wid], idx_v)
        base = wid * per_w

        def gather_start(j, b):
            pltpu.async_copy(table_hbm.at[idx_v.at[j]], rows_v.at[b], gsem[b])

        def gather_wait(j, b):
            pltpu.make_async_copy(table_hbm.at[idx_v.at[j]], rows_v.at[b], gsem[b]).wait()

        def wb_start(j, b):
            pltpu.async_copy(rows_v.at[b], out_hbm.at[pl.ds(base + j * CH, CH)], wsem[b])

        def wb_wait(b):
            pltpu.make_async_copy(
                rows_v.at[b], out_hbm.at[pl.ds(base, CH)], wsem[b]
            ).wait()

        for b in range(K):
            gather_start(b, b)

        for b in range(R):
            jn = b + K
            if jn >= R:
                wb_wait(jn % R)
            gather_start(jn, jn % R)
            gather_wait(b, b)
            wb_start(b, b)

        def group(g, carry):
            i0 = g * R
            for b in range(R):
                bn = (b + K) % R
                wb_wait(bn)
                gather_start(i0 + b + K, bn)
                gather_wait(i0 + b, b)
                wb_start(i0 + b, b)
            return carry

        lax.fori_loop(1, n_groups - 1, group, 0)

        i0 = (n_groups - 1) * R
        for b in range(R):
            jn = i0 + b + K
            if jn < n_ch:
                bn = (b + K) % R
                wb_wait(bn)
                gather_start(jn, bn)
            gather_wait(i0 + b, b)
            wb_start(i0 + b, b)

        for b in range(R):
            wb_wait(b)

    return k(idx, lut)


def kernel(x, lut):
    b, h = x.shape
    v, d = lut.shape
    n = b * h
    assert n % (NW * CH) == 0
    n_ch = n // (NW * CH)
    idx = x.reshape(NW, n_ch, CH)
    out = _gather(idx, lut, n_ch, d)
    return out.reshape(b, h, d)


# speedup vs baseline: 1.1442x; 1.0264x over previous
import functools

import jax
import jax.numpy as jnp
from jax import lax
from jax.experimental import pallas as pl
from jax.experimental.pallas import tpu as pltpu
from jax.experimental.pallas import tpu_sc as plsc

_INFO = plsc.get_sparse_core_info()
NC = _INFO.num_cores
NS = _INFO.num_subcores
NW = NC * NS

CH = 128
R = 8
K = 4


@functools.partial(jax.jit, static_argnames=("n_ch", "d"))
def _gather(idx, lut, n_ch, d):
    n = NW * n_ch * CH
    per_w = n_ch * CH
    n_groups = n_ch // R
    assert n_ch % R == 0 and n_groups >= 3
    mesh = plsc.VectorSubcoreMesh(core_axis_name="c", subcore_axis_name="s")

    @functools.partial(
        pl.kernel,
        out_type=jax.ShapeDtypeStruct((n, d), jnp.float32),
        mesh=mesh,
        scratch_types=(
            [pltpu.VMEM((n_ch, CH), jnp.int32), pltpu.VMEM((R, CH, d), jnp.float32)]
            + [pltpu.SemaphoreType.DMA] * (2 * R)
        ),
        compiler_params=pltpu.CompilerParams(use_tc_tiling_on_sc=False),
    )
    def k(idx_hbm, table_hbm, out_hbm, idx_v, rows_v, *sems):
        gsem = sems[:R]
        wsem = sems[R:]
        cid = lax.axis_index("c")
        sid = lax.axis_index("s")
        wid = sid * NC + cid
        pltpu.sync_copy(idx_hbm.at[wid], idx_v)
        base = wid * per_w

        def gather_start(j, b):
            pltpu.async_copy(table_hbm.at[idx_v.at[j]], rows_v.at[b], gsem[b])

        def gather_wait(j, b):
            pltpu.make_async_copy(table_hbm.at[idx_v.at[j]], rows_v.at[b], gsem[b]).wait()

        def wb_start(j, b):
            pltpu.async_copy(rows_v.at[b], out_hbm.at[pl.ds(base + j * CH, CH)], wsem[b])

        def wb_wait(b):
            pltpu.make_async_copy(
                rows_v.at[b], out_hbm.at[pl.ds(base, CH)], wsem[b]
            ).wait()

        for b in range(K):
            gather_start(b, b)

        for b in range(R):
            jn = b + K
            if jn >= R:
                wb_wait(jn % R)
            gather_start(jn, jn % R)
            gather_wait(b, b)
            wb_start(b, b)

        def group(g, carry):
            i0 = g * R
            for b in range(R):
                bn = (b + K) % R
                wb_wait(bn)
                gather_start(i0 + b + K, bn)
                gather_wait(i0 + b, b)
                wb_start(i0 + b, b)
            return carry

        lax.fori_loop(1, n_groups - 1, group, 0)

        i0 = (n_groups - 1) * R
        for b in range(R):
            jn = i0 + b + K
            if jn < n_ch:
                bn = (b + K) % R
                wb_wait(bn)
                gather_start(jn, bn)
            gather_wait(i0 + b, b)
            wb_start(i0 + b, b)

        for b in range(R):
            wb_wait(b)

    return k(idx, lut)


def kernel(x, lut):
    b, h = x.shape
    v, d = lut.shape
    n = b * h
    assert n % (NW * CH) == 0
    n_ch = n // (NW * CH)
    idx = x.T.reshape(NW, n_ch, CH)
    out = _gather(idx, lut, n_ch, d)
    return out.reshape(h, b, d).transpose(1, 0, 2)
